# baseline (speedup 1.0000x reference)
import functools

import jax
import jax.numpy as jnp
from jax import lax
from jax.experimental import pallas as pl
from jax.experimental.pallas import tpu as pltpu
from jax.experimental.pallas import tpu_sc as plsc

BATCH = 16384
TEXT_DIM = 128
FINAL_DIM = 64
HIDDEN = (TEXT_DIM + 1) // 2

NUM_CORES = 2
NUM_SUBCORES = 16
NW = NUM_CORES * NUM_SUBCORES
BPW = BATCH // NW
CHUNK = 128
NCH = BPW // CHUNK
IDS_COLS = 128


def _sc_gathers(user_ids2d, item_ids2d, user_table, item_text_table):
  mesh = plsc.VectorSubcoreMesh(core_axis_name="c", subcore_axis_name="s")

  @functools.partial(
      pl.kernel,
      out_type=(
          jax.ShapeDtypeStruct((BATCH, FINAL_DIM), jnp.float32),
          jax.ShapeDtypeStruct((BATCH, TEXT_DIM), jnp.float32),
      ),
      mesh=mesh,
      compiler_params=pltpu.CompilerParams(use_tc_tiling_on_sc=False),
      scratch_types=[
          pltpu.VMEM((NCH, CHUNK), jnp.int32),
          pltpu.VMEM((NCH, CHUNK), jnp.int32),
          pltpu.VMEM((BPW, FINAL_DIM), jnp.float32),
          pltpu.VMEM((BPW, TEXT_DIM), jnp.float32),
          pltpu.SemaphoreType.DMA,
      ],
  )
  def k(uids_hbm, iids_hbm, utab_hbm, itab_hbm, uout_hbm, tout_hbm,
        uidx, iidx, urows, irows, sem):
    wid = lax.axis_index("s") * NUM_CORES + lax.axis_index("c")
    row0 = wid * NCH
    pltpu.sync_copy(uids_hbm.at[pl.ds(row0, NCH)], uidx)
    pltpu.sync_copy(iids_hbm.at[pl.ds(row0, NCH)], iidx)
    handles = []
    for j in range(NCH):
      handles.append(pltpu.async_copy(
          utab_hbm.at[uidx.at[j]], urows.at[pl.ds(j * CHUNK, CHUNK)], sem))
      handles.append(pltpu.async_copy(
          itab_hbm.at[iidx.at[j]], irows.at[pl.ds(j * CHUNK, CHUNK)], sem))
    for h in handles:
      h.wait()
    base = wid * BPW
    pltpu.sync_copy(urows, uout_hbm.at[pl.ds(base, BPW)])
    pltpu.sync_copy(irows, tout_hbm.at[pl.ds(base, BPW)])

  return k(user_ids2d, item_ids2d, user_table, item_text_table)


def _mlp_body(x_ref, p_ref, w1m_ref, w1l_ref, b1_ref, w2t_ref, b2_ref, o_ref):
  h = jnp.dot(x_ref[...], w1m_ref[...], preferred_element_type=jnp.float32)
  h = h + p_ref[...] * w1l_ref[...] + b1_ref[...]
  h = jnp.maximum(h, 0.0)
  o_ref[...] = (
      jnp.dot(h, w2t_ref[...], preferred_element_type=jnp.float32)
      + b2_ref[...])


def _mlp(text_vecs, prices_col, w1m, w1l, b1r, w2t, b2r, block_m=2048):
  grid = (BATCH // block_m,)
  return pl.pallas_call(
      _mlp_body,
      grid=grid,
      in_specs=[
          pl.BlockSpec((block_m, TEXT_DIM), lambda i: (i, 0)),
          pl.BlockSpec((block_m, 1), lambda i: (i, 0)),
          pl.BlockSpec((TEXT_DIM, HIDDEN), lambda i: (0, 0)),
          pl.BlockSpec((1, HIDDEN), lambda i: (0, 0)),
          pl.BlockSpec((1, HIDDEN), lambda i: (0, 0)),
          pl.BlockSpec((HIDDEN, FINAL_DIM), lambda i: (0, 0)),
          pl.BlockSpec((1, FINAL_DIM), lambda i: (0, 0)),
      ],
      out_specs=pl.BlockSpec((block_m, FINAL_DIM), lambda i: (i, 0)),
      out_shape=jax.ShapeDtypeStruct((BATCH, FINAL_DIM), jnp.float32),
  )(text_vecs, prices_col, w1m, w1l, b1r, w2t, b2r)


def kernel(user_ids, item_ids, item_prices, user_table, item_text_table,
           W1, b1, W2, b2):
  uids2 = user_ids.astype(jnp.int32).reshape(BATCH // IDS_COLS, IDS_COLS)
  iids2 = item_ids.astype(jnp.int32).reshape(BATCH // IDS_COLS, IDS_COLS)
  user_vec, text_vecs = _sc_gathers(uids2, iids2, user_table, item_text_table)
  w1m = W1[:, :TEXT_DIM].T
  w1l = W1[:, TEXT_DIM:].T
  item_vec = _mlp(text_vecs, item_prices.reshape(BATCH, 1), w1m, w1l,
                  b1.reshape(1, HIDDEN), W2.T, b2.reshape(1, FINAL_DIM))
  return user_vec, item_vec


# speedup vs baseline: 1.6467x; 1.6467x over previous
import functools

import jax
import jax.numpy as jnp
from jax import lax
from jax.experimental import pallas as pl
from jax.experimental.pallas import tpu as pltpu
from jax.experimental.pallas import tpu_sc as plsc

BATCH = 16384
TEXT_DIM = 128
FINAL_DIM = 64
HIDDEN = (TEXT_DIM + 1) // 2

NUM_CORES = 2
NUM_SUBCORES = 16
NW = NUM_CORES * NUM_SUBCORES
BPW = BATCH // NW
CHUNK = 128
NCH = BPW // CHUNK


def _sc_gathers(user_ids2d, item_ids2d, user_table, item_text_table):
  mesh = plsc.VectorSubcoreMesh(core_axis_name="c", subcore_axis_name="s")

  @functools.partial(
      pl.kernel,
      out_type=(
          jax.ShapeDtypeStruct((BATCH, FINAL_DIM), jnp.float32),
          jax.ShapeDtypeStruct((BATCH, TEXT_DIM), jnp.float32),
      ),
      mesh=mesh,
      scratch_types=[
          pltpu.VMEM((NCH, CHUNK), jnp.int32),
          pltpu.VMEM((1, BPW), jnp.int32),
          pltpu.VMEM((BPW, FINAL_DIM), jnp.float32),
          pltpu.VMEM((CHUNK, TEXT_DIM), jnp.float32),
          pltpu.SemaphoreType.DMA,
          pltpu.SemaphoreType.DMA,
      ],
  )
  def k(uids_hbm, iids_hbm, utab_hbm, itab_hbm, uout_hbm, tout_hbm,
        iidx, uidx_v, urows, irows, isem, usem):
    wid = lax.axis_index("s") * NUM_CORES + lax.axis_index("c")
    row0 = wid * NCH
    pltpu.sync_copy(iids_hbm.at[pl.ds(row0, NCH)], iidx)
    pltpu.sync_copy(uids_hbm.at[pl.ds(wid, 1)], uidx_v)

    def row_dma_group(g, carry):
      v = uidx_v[0, pl.ds(g * 16, 16)]
      base16 = g * 16
      for lane in range(16):
        r = v[lane]
        pltpu.async_copy(
            utab_hbm.at[pl.ds(r, 1)], urows.at[pl.ds(base16 + lane, 1)], usem)
      return carry

    lax.fori_loop(0, BPW // 16, row_dma_group, 0)
    base = wid * BPW
    for j in range(NCH):
      pltpu.async_copy(itab_hbm.at[iidx.at[j]], irows, isem).wait()
      pltpu.sync_copy(irows, tout_hbm.at[pl.ds(base + j * CHUNK, CHUNK)])
    pltpu.make_async_copy(utab_hbm.at[pl.ds(0, BPW)], urows, usem).wait()
    pltpu.sync_copy(urows, uout_hbm.at[pl.ds(base, BPW)])

  return k(user_ids2d, item_ids2d, user_table, item_text_table)


def _mlp_body(x_ref, p_ref, w1m_ref, w1l_ref, b1_ref, w2t_ref, b2_ref, o_ref):
  h = jnp.dot(x_ref[...], w1m_ref[...], preferred_element_type=jnp.float32)
  h = h + p_ref[...] * w1l_ref[...] + b1_ref[...]
  h = jnp.maximum(h, 0.0)
  o_ref[...] = (
      jnp.dot(h, w2t_ref[...], preferred_element_type=jnp.float32)
      + b2_ref[...])


def _mlp(text_vecs, prices_col, w1m, w1l, b1r, w2t, b2r, block_m=2048):
  grid = (BATCH // block_m,)
  return pl.pallas_call(
      _mlp_body,
      grid=grid,
      in_specs=[
          pl.BlockSpec((block_m, TEXT_DIM), lambda i: (i, 0)),
          pl.BlockSpec((block_m, 1), lambda i: (i, 0)),
          pl.BlockSpec((TEXT_DIM, HIDDEN), lambda i: (0, 0)),
          pl.BlockSpec((1, HIDDEN), lambda i: (0, 0)),
          pl.BlockSpec((1, HIDDEN), lambda i: (0, 0)),
          pl.BlockSpec((HIDDEN, FINAL_DIM), lambda i: (0, 0)),
          pl.BlockSpec((1, FINAL_DIM), lambda i: (0, 0)),
      ],
      out_specs=pl.BlockSpec((block_m, FINAL_DIM), lambda i: (i, 0)),
      out_shape=jax.ShapeDtypeStruct((BATCH, FINAL_DIM), jnp.float32),
  )(text_vecs, prices_col, w1m, w1l, b1r, w2t, b2r)


def kernel(user_ids, item_ids, item_prices, user_table, item_text_table,
           W1, b1, W2, b2):
  uids2 = user_ids.astype(jnp.int32).reshape(NW, BPW)
  iids2 = item_ids.astype(jnp.int32).reshape(BATCH // CHUNK, CHUNK)
  user_vec, text_vecs = _sc_gathers(uids2, iids2, user_table, item_text_table)
  w1m = W1[:, :TEXT_DIM].T
  w1l = W1[:, TEXT_DIM:].T
  item_vec = _mlp(text_vecs, item_prices.reshape(BATCH, 1), w1m, w1l,
                  b1.reshape(1, HIDDEN), W2.T, b2.reshape(1, FINAL_DIM))
  return user_vec, item_vec
